# Initial kernel scaffold; baseline (speedup 1.0000x reference)
import jax
import jax.numpy as jnp
from jax.experimental import pallas as pl


def kernel(hidden_state, gate_w, gate_proj_w, up_proj_w, down_proj_w):
    raise NotImplementedError("write your pallas kernel here")


# speedup vs baseline: 1.0436x; 1.0436x over previous
import functools

import jax
import jax.numpy as jnp
from jax.experimental import pallas as pl
from jax.experimental.pallas import tpu as pltpu

E = 8
TOP_K = 2
H = 1024
I = 2048

TT = 512
IT = 512


def _router_body(x_ref, gw_ref, logits_ref, w_ref):
    x = x_ref[...]
    gw = gw_ref[...]
    logits = jax.lax.dot_general(
        x, gw, (((1,), (1,)), ((), ())),
        preferred_element_type=jnp.float32)
    logits_ref[...] = logits
    m = jnp.max(logits, axis=-1, keepdims=True)
    ex = jnp.exp(logits - m)
    p = ex / jnp.sum(ex, axis=-1, keepdims=True)
    lane = jax.lax.broadcasted_iota(jnp.int32, p.shape, 1)
    m1 = jnp.max(p, axis=-1, keepdims=True)
    i1 = jnp.min(jnp.where(p == m1, lane, E), axis=-1, keepdims=True)
    mask1 = lane == i1
    p2 = jnp.where(mask1, -1.0, p)
    m2 = jnp.max(p2, axis=-1, keepdims=True)
    i2 = jnp.min(jnp.where(p2 == m2, lane, E), axis=-1, keepdims=True)
    mask2 = lane == i2
    w_ref[...] = jnp.where(mask1 | mask2, p, 0.0)


def _moe_body(x_ref, gw_ref, uw_ref, dw_ref, w_ref, out_ref, acc_ref):
    e = pl.program_id(1)
    i = pl.program_id(2)

    @pl.when((e == 0) & (i == 0))
    def _():
        acc_ref[...] = jnp.zeros_like(acc_ref)

    x = x_ref[...]
    gw = gw_ref[0]
    uw = uw_ref[0]
    dw = dw_ref[0]
    g = jax.lax.dot_general(x, gw, (((1,), (1,)), ((), ())),
                            preferred_element_type=jnp.float32)
    u = jax.lax.dot_general(x, uw, (((1,), (1,)), ((), ())),
                            preferred_element_type=jnp.float32)
    h = (g * jax.nn.sigmoid(g)) * u
    o = jax.lax.dot_general(h, dw, (((1,), (1,)), ((), ())),
                            preferred_element_type=jnp.float32)
    wfull = w_ref[...]
    lane = jax.lax.broadcasted_iota(jnp.int32, wfull.shape, 1)
    w = jnp.sum(jnp.where(lane == e, wfull, 0.0), axis=-1, keepdims=True)
    acc_ref[...] += w * o

    @pl.when((e == E - 1) & (i == pl.num_programs(2) - 1))
    def _():
        out_ref[...] = acc_ref[...]


@functools.partial(jax.jit, static_argnames=("interpret",))
def _run(x, gate_w, gate_proj_w, up_proj_w, down_proj_w, interpret=False):
    T = x.shape[0]
    NT = T // TT
    NI = I // IT

    logits, w = pl.pallas_call(
        _router_body,
        grid=(NT,),
        in_specs=[
            pl.BlockSpec((TT, H), lambda t: (t, 0)),
            pl.BlockSpec((E, H), lambda t: (0, 0)),
        ],
        out_specs=[
            pl.BlockSpec((TT, E), lambda t: (t, 0)),
            pl.BlockSpec((TT, E), lambda t: (t, 0)),
        ],
        out_shape=[
            jax.ShapeDtypeStruct((T, E), jnp.float32),
            jax.ShapeDtypeStruct((T, E), jnp.float32),
        ],
        interpret=interpret,
    )(x, gate_w)

    final = pl.pallas_call(
        _moe_body,
        grid=(NT, E, NI),
        in_specs=[
            pl.BlockSpec((TT, H), lambda t, e, i: (t, 0)),
            pl.BlockSpec((1, IT, H), lambda t, e, i: (e, i, 0)),
            pl.BlockSpec((1, IT, H), lambda t, e, i: (e, i, 0)),
            pl.BlockSpec((1, H, IT), lambda t, e, i: (e, 0, i)),
            pl.BlockSpec((TT, E), lambda t, e, i: (t, 0)),
        ],
        out_specs=pl.BlockSpec((TT, H), lambda t, e, i: (t, 0)),
        out_shape=jax.ShapeDtypeStruct((T, H), jnp.float32),
        scratch_shapes=[pltpu.VMEM((TT, H), jnp.float32)],
        compiler_params=pltpu.CompilerParams(
            dimension_semantics=("arbitrary", "arbitrary", "arbitrary")),
        interpret=interpret,
    )(x, gate_proj_w, up_proj_w, down_proj_w, w)

    return final, logits


def kernel(hidden_state, gate_w, gate_proj_w, up_proj_w, down_proj_w):
    b, s, h = hidden_state.shape
    x = hidden_state.reshape(-1, h)
    final, logits = _run(x, gate_w, gate_proj_w, up_proj_w, down_proj_w)
    return final.reshape(b, s, h), logits
